# baseline (speedup 1.0000x reference)
import functools

import jax
import jax.numpy as jnp
from jax import lax
from jax.experimental import pallas as pl
from jax.experimental.pallas import tpu as pltpu
from jax.experimental.pallas import tpu_sc as plsc

VOCAB = 50304
D = 1024
N = 4 * 2048

_info = plsc.get_sparse_core_info()
NC, NS = _info.num_cores, _info.num_subcores
NW = NC * NS
PER_W = N // NW
CHUNK = 32
NCHUNK = PER_W // CHUNK

NBUF = 3

_mesh = plsc.VectorSubcoreMesh(core_axis_name="c", subcore_axis_name="s")


@functools.partial(
    pl.kernel,
    out_type=tuple(
        jax.ShapeDtypeStruct((N, D), jnp.float32) for _ in range(12)
    ),
    mesh=_mesh,
    scratch_types=[
        pltpu.VMEM((NCHUNK, CHUNK), jnp.int32),
        tuple(pltpu.VMEM((CHUNK, D), jnp.float32) for _ in range(NBUF)),
        tuple(pltpu.SemaphoreType.DMA for _ in range(NBUF)),
        tuple(pltpu.SemaphoreType.DMA for _ in range(NBUF)),
        pltpu.SemaphoreType.DMA,
    ],
)
def _gather6(idx_hbm, w0, w1, w2, w3, w4, w5,
             o0, o1, o2, o3, o4, o5, o6, o7, o8, o9, o10, o11,
             idx_v, bufs, gsems, wsems, hsem):
    wid = lax.axis_index("s") * NC + lax.axis_index("c")
    base = wid * PER_W
    for c in range(NCHUNK):
        pltpu.sync_copy(idx_hbm.at[pl.ds(base + c * CHUNK, CHUNK)],
                        idx_v.at[c])
    outs = (o0, o1, o2, o3, o4, o5, o6, o7, o8, o9, o10, o11)
    tasks = [(w, outs[t], outs[11 - t], c)
             for t, w in enumerate((w0, w1, w2, w3, w4, w5))
             for c in range(NCHUNK)]
    nt = len(tasks)
    gdesc = [None] * NBUF
    wdesc = [None] * NBUF
    hdescs = []
    for i in range(nt + 1):
        if i >= 1:
            k = i - 1
            s = k % NBUF
            _, oa, ob, c = tasks[k]
            gdesc[s].wait()
            dst = pl.ds(base + c * CHUNK, CHUNK)
            wdesc[s] = (pltpu.async_copy(bufs[s], oa.at[dst], wsems[s]),
                        oa, ob, dst)
        if i < nt:
            s = i % NBUF
            w, _, _, c = tasks[i]
            if wdesc[s] is not None:
                d, oa, ob, dst = wdesc[s]
                d.wait()
                hdescs.append(
                    pltpu.async_copy(oa.at[dst], ob.at[dst], hsem))
            gdesc[s] = pltpu.async_copy(
                w.at[idx_v.at[c]], bufs[s], gsems[s])
    for ent in wdesc:
        if ent is not None:
            d, oa, ob, dst = ent
            d.wait()
            hdescs.append(pltpu.async_copy(oa.at[dst], ob.at[dst], hsem))
    for d in hdescs:
        d.wait()


def kernel(inputs, W0, W1, W2, W3, W4, W5):
    B, S = inputs.shape
    flat = inputs.reshape(N)
    outs = _gather6(flat, W0, W1, W2, W3, W4, W5)
    return tuple(o.reshape(B, S, D) for o in outs)


# speedup vs baseline: 25.0032x; 25.0032x over previous
import functools

import jax
import jax.numpy as jnp
from jax import lax
from jax.experimental import pallas as pl
from jax.experimental.pallas import tpu as pltpu
from jax.experimental.pallas import tpu_sc as plsc

VOCAB = 50304
D = 1024
N = 4 * 2048

_info = plsc.get_sparse_core_info()
NC, NS = _info.num_cores, _info.num_subcores
NW = NC * NS
PER_W = N // NW
CHUNK = 32
NCHUNK = PER_W // CHUNK

NBUF = 3

_mesh = plsc.VectorSubcoreMesh(core_axis_name="c", subcore_axis_name="s")


@functools.partial(
    pl.kernel,
    out_type=tuple(
        jax.ShapeDtypeStruct((N, D), jnp.float32) for _ in range(12)
    ),
    mesh=_mesh,
    scratch_types=[
        pltpu.VMEM((NCHUNK, CHUNK), jnp.int32),
        tuple(pltpu.VMEM((CHUNK, D), jnp.float32) for _ in range(NBUF)),
        tuple(pltpu.SemaphoreType.DMA for _ in range(NBUF)),
        tuple(pltpu.SemaphoreType.DMA for _ in range(NBUF)),
    ],
)
def _gather6(idx_hbm, w0, w1, w2, w3, w4, w5,
             o0, o1, o2, o3, o4, o5, o6, o7, o8, o9, o10, o11,
             idx_v, bufs, gsems, wsems):
    wid = lax.axis_index("s") * NC + lax.axis_index("c")
    base = wid * PER_W
    pltpu.sync_copy(idx_hbm.at[wid], idx_v)
    outs = (o0, o1, o2, o3, o4, o5, o6, o7, o8, o9, o10, o11)
    tasks = [(w, outs[t], outs[11 - t], c)
             for t, w in enumerate((w0, w1, w2, w3, w4, w5))
             for c in range(NCHUNK)]
    nt = len(tasks)
    gdesc = [None] * NBUF
    wdesc = [None] * NBUF
    for i in range(nt + 1):
        if i >= 1:
            k = i - 1
            s = k % NBUF
            _, oa, ob, c = tasks[k]
            gdesc[s].wait()
            dst = pl.ds(base + c * CHUNK, CHUNK)
            wdesc[s] = (
                pltpu.async_copy(bufs[s], oa.at[dst], wsems[s]),
                pltpu.async_copy(bufs[s], ob.at[dst], wsems[s]),
            )
        if i < nt:
            s = i % NBUF
            w, _, _, c = tasks[i]
            if wdesc[s] is not None:
                for d in wdesc[s]:
                    d.wait()
            gdesc[s] = pltpu.async_copy(
                w.at[idx_v.at[c]], bufs[s], gsems[s])
    for pair in wdesc:
        for d in pair:
            d.wait()


def kernel(inputs, W0, W1, W2, W3, W4, W5):
    B, S = inputs.shape
    idx3 = inputs.reshape(NW, NCHUNK, CHUNK)
    outs = _gather6(idx3, W0, W1, W2, W3, W4, W5)
    return tuple(o.reshape(B, S, D) for o in outs)
